# baseline (speedup 1.0000x reference)
import functools

import jax
import jax.numpy as jnp
from jax import lax
from jax.experimental import pallas as pl
from jax.experimental.pallas import tpu as pltpu
from jax.experimental.pallas import tpu_sc as plsc

_NUM_CORES = 2
_NUM_SUBCORES = 16
_NW = _NUM_CORES * _NUM_SUBCORES
_LANES = 16
_WIDE = 128


@functools.cache
def _build_repack(n_rows: int, dim: int):
    pairs = n_rows // 2
    chunk = (n_rows // _NW) // 16 * 16
    tail_base = chunk * _NW
    tail = n_rows - tail_base
    mesh = plsc.VectorSubcoreMesh(core_axis_name="c", subcore_axis_name="s")

    @functools.partial(
        pl.kernel,
        mesh=mesh,
        out_type=jax.ShapeDtypeStruct((pairs, _WIDE), jnp.float32),
    )
    def repack_kernel(table_hbm, wide_hbm):
        wid = lax.axis_index("s") * _NUM_CORES + lax.axis_index("c")
        base = wid * chunk
        narrow = wide_hbm.reshape(pairs * 2, dim)
        pltpu.sync_copy(
            table_hbm.at[pl.ds(base, chunk)],
            narrow.at[pl.ds(base, chunk)],
        )
        if tail:
            @pl.when(wid == _NW - 1)
            def _():
                pltpu.sync_copy(
                    table_hbm.at[pl.ds(tail_base, tail)],
                    narrow.at[pl.ds(tail_base, tail)],
                )

    return repack_kernel


@functools.cache
def _build_gather(batch: int, pairs: int, dim: int):
    b_per_w = batch // _NW
    n_groups = b_per_w // _LANES
    mesh = plsc.VectorSubcoreMesh(core_axis_name="c", subcore_axis_name="s")

    @functools.partial(
        pl.kernel,
        mesh=mesh,
        out_type=jax.ShapeDtypeStruct((batch, dim), jnp.float32),
        scratch_types=[
            pltpu.VMEM((b_per_w,), jnp.int32),
            pltpu.VMEM((b_per_w,), jnp.int32),
            pltpu.VMEM((b_per_w // 2, _WIDE), jnp.float32),
            pltpu.VMEM((b_per_w, dim), jnp.float32),
            pltpu.SemaphoreType.DMA,
        ],
    )
    def gather_kernel(idx_hbm, wide_hbm, out_hbm, idx_v, pid_v, gbuf, rows_v, sem):
        wid = lax.axis_index("s") * _NUM_CORES + lax.axis_index("c")
        base = wid * b_per_w
        half = b_per_w // 2
        pltpu.sync_copy(idx_hbm.at[pl.ds(base, b_per_w)], idx_v)
        for g in range(n_groups):
            vec = idx_v[pl.ds(g * _LANES, _LANES)]
            pid_v[pl.ds(g * _LANES, _LANES)] = jnp.right_shift(vec, 1)
        for h in range(2):
            pltpu.async_copy(
                wide_hbm.at[pid_v.at[pl.ds(h * half, half)]], gbuf, sem
            ).wait()
            for g in range(n_groups // 2):
                vec = idx_v[pl.ds(h * half + g * _LANES, _LANES)]
                voff = jnp.bitwise_and(vec, 1) * dim
                for k in range(_LANES):
                    r = g * _LANES + k
                    off = voff[k]
                    for j in range(dim // _LANES):
                        rows_v[h * half + r, pl.ds(j * _LANES, _LANES)] = gbuf[
                            r, pl.ds(off + j * _LANES, _LANES)
                        ]
        pltpu.sync_copy(rows_v, out_hbm.at[pl.ds(base, b_per_w)])

    return gather_kernel


def kernel(labels, table):
    labels = labels.astype(jnp.int32)
    batch = labels.shape[0]
    dim = table.shape[1]
    table = table.astype(jnp.float32)
    n_rows = (table.shape[0] - 1) // 2 * 2
    wide = table[:n_rows].reshape(n_rows // 2, 2 * dim)
    return _build_gather(batch, n_rows // 2, dim)(labels, wide)


# speedup vs baseline: 1.7470x; 1.7470x over previous
import functools

import jax
import jax.numpy as jnp
from jax import lax
from jax.experimental import pallas as pl
from jax.experimental.pallas import tpu as pltpu
from jax.experimental.pallas import tpu_sc as plsc

_NUM_CORES = 2
_NUM_SUBCORES = 16
_NW = _NUM_CORES * _NUM_SUBCORES
_LANES = 16
_NSEM = 4


@functools.cache
def _build_gather(batch: int, dim: int):
    b_per_w = batch // _NW
    n_groups = b_per_w // _LANES
    groups_per_sem = n_groups // _NSEM
    mesh = plsc.VectorSubcoreMesh(core_axis_name="c", subcore_axis_name="s")

    @functools.partial(
        pl.kernel,
        mesh=mesh,
        out_type=jax.ShapeDtypeStruct((batch, dim), jnp.float32),
        scratch_types=[
            pltpu.VMEM((b_per_w,), jnp.int32),
            pltpu.VMEM((b_per_w, dim), jnp.float32),
        ]
        + [pltpu.SemaphoreType.DMA] * _NSEM,
    )
    def gather_kernel(idx_hbm, table_hbm, out_hbm, idx_v, rows_v, *sems):
        wid = lax.axis_index("s") * _NUM_CORES + lax.axis_index("c")
        base = wid * b_per_w
        pltpu.sync_copy(idx_hbm.at[pl.ds(base, b_per_w)], idx_v)

        @plsc.parallel_loop(0, n_groups // _NSEM, 1, unroll=2)
        def _(gg):
            for s in range(_NSEM):
                g = gg * _NSEM + s
                vec = idx_v[pl.ds(g * _LANES, _LANES)]
                for lane in range(_LANES):
                    pltpu.async_copy(
                        table_hbm.at[vec[lane]],
                        rows_v.at[g * _LANES + lane],
                        sems[s],
                    )

        rows_per_sem = groups_per_sem * _LANES
        for s in range(_NSEM):
            pltpu.make_async_copy(
                table_hbm.at[pl.ds(0, rows_per_sem)],
                rows_v.at[pl.ds(0, rows_per_sem)],
                sems[s],
            ).wait()
        pltpu.sync_copy(rows_v, out_hbm.at[pl.ds(base, b_per_w)])

    return gather_kernel


def kernel(labels, table):
    labels = labels.astype(jnp.int32)
    batch = labels.shape[0]
    dim = table.shape[1]
    table = table.astype(jnp.float32)
    return _build_gather(batch, dim)(labels, table)
